# baseline (speedup 1.0000x reference)
import functools

import jax
import jax.numpy as jnp
from jax import lax
from jax.experimental import pallas as pl
from jax.experimental.pallas import tpu as pltpu
from jax.experimental.pallas import tpu_sc as plsc

N_IDS = 1000000
EMBED_DIM = 32
BATCH = 16384
HIST = 50

NC = 2
NS = 16
NW = NC * NS
ROWS_PER_W = BATCH // NW
ROWS_PER_CHUNK = 2
CHUNK = ROWS_PER_CHUNK * HIST
NCHUNKS = ROWS_PER_W // ROWS_PER_CHUNK


def _sc_embedding_sum(idx3, table):
  mesh = plsc.VectorSubcoreMesh(core_axis_name="c", subcore_axis_name="s")

  @functools.partial(
      pl.kernel,
      mesh=mesh,
      out_type=jax.ShapeDtypeStruct((BATCH, EMBED_DIM), jnp.float32),
      compiler_params=pltpu.CompilerParams(use_tc_tiling_on_sc=False),
      scratch_types=[
          pltpu.VMEM((NCHUNKS, CHUNK), jnp.int32),
          pltpu.VMEM((CHUNK, EMBED_DIM), jnp.float32),
          pltpu.VMEM((CHUNK, EMBED_DIM), jnp.float32),
          pltpu.VMEM((CHUNK, EMBED_DIM), jnp.float32),
          pltpu.VMEM((CHUNK, EMBED_DIM), jnp.float32),
          pltpu.VMEM((ROWS_PER_W, EMBED_DIM), jnp.float32),
          pltpu.SemaphoreType.DMA,
          pltpu.SemaphoreType.DMA,
          pltpu.SemaphoreType.DMA,
          pltpu.SemaphoreType.DMA,
          pltpu.SemaphoreType.DMA,
          pltpu.SemaphoreType.DMA,
      ],
  )
  def k(idx_hbm, table_hbm, out_hbm, idx_v, buf0, buf1, buf2, buf3, out_v,
        sem0, sem1, sem2, sem3, isem, osem):
    bufs = (buf0, buf1, buf2, buf3)
    sems = (sem0, sem1, sem2, sem3)
    nbuf = 4
    wid = lax.axis_index("s") * NC + lax.axis_index("c")

    head = 32
    pltpu.sync_copy(idx_hbm.at[wid, pl.ds(0, head)], idx_v.at[pl.ds(0, head)])
    pltpu.async_copy(idx_hbm.at[wid, pl.ds(head, NCHUNKS - head)],
                     idx_v.at[pl.ds(head, NCHUNKS - head)], isem)

    def start(c, buf, sem):
      pltpu.async_copy(table_hbm.at[idx_v.at[c]], buf, sem)

    def wait(buf, sem):
      pltpu.make_async_copy(table_hbm.at[idx_v.at[0]], buf, sem).wait()

    def accumulate(buf, local_row0):
      for g in range(ROWS_PER_CHUNK):
        base = g * HIST
        a0 = buf[base, pl.ds(0, 16)]
        a1 = buf[base, pl.ds(16, 16)]
        for j in range(1, HIST):
          a0 = a0 + buf[base + j, pl.ds(0, 16)]
          a1 = a1 + buf[base + j, pl.ds(16, 16)]
        out_v[local_row0 + g, pl.ds(0, 16)] = a0
        out_v[local_row0 + g, pl.ds(16, 16)] = a1

    for c in range(nbuf - 1):
      start(c, bufs[c], sems[c])

    def body(i, _):
      for k in range(nbuf):
        c = nbuf * i + k
        ahead = c + nbuf - 1

        @pl.when(ahead < NCHUNKS)
        def _():
          start(ahead, bufs[(k + nbuf - 1) % nbuf], sems[(k + nbuf - 1) % nbuf])

        wait(bufs[k], sems[k])
        accumulate(bufs[k], ROWS_PER_CHUNK * c)

      @pl.when(i == 6)
      def _():
        pltpu.make_async_copy(
            idx_hbm.at[wid, pl.ds(head, NCHUNKS - head)],
            idx_v.at[pl.ds(head, NCHUNKS - head)], isem).wait()

      @pl.when(i == (NCHUNKS // nbuf) // 2)
      def _():
        pltpu.async_copy(
            out_v.at[pl.ds(0, ROWS_PER_W // 2)],
            out_hbm.at[pl.ds(wid * ROWS_PER_W, ROWS_PER_W // 2)], osem)
      return 0

    lax.fori_loop(0, NCHUNKS // nbuf, body, 0)

    pltpu.sync_copy(
        out_v.at[pl.ds(ROWS_PER_W // 2, ROWS_PER_W // 2)],
        out_hbm.at[pl.ds(wid * ROWS_PER_W + ROWS_PER_W // 2, ROWS_PER_W // 2)])
    pltpu.make_async_copy(
        out_v.at[pl.ds(0, ROWS_PER_W // 2)],
        out_hbm.at[pl.ds(wid * ROWS_PER_W, ROWS_PER_W // 2)], osem).wait()

  return k(idx3, table)


def kernel(inputs, W):
  idx3 = inputs.astype(jnp.int32).reshape(NW, NCHUNKS, CHUNK)
  return _sc_embedding_sum(idx3, W)
